# baseline (speedup 1.0000x reference)
import functools

import jax
import jax.numpy as jnp
from jax.experimental import pallas as pl
from jax.experimental.pallas import tpu as pltpu
from jax.experimental.pallas import tpu_sc as plsc

B, T, V, E = 32, 12, 17, 38
HEADS, HC, HL, NCLS = 8, 128, 512, 500
G = 2 * T
NGV = G * V
NGE = G * E
HID = HEADS * HC
D = HC * (V + 2)
GP = 24
NPV = G * GP
F32 = jnp.float32
HI = jax.lax.Precision.HIGHEST
BF16 = jnp.bfloat16
_DN_NT = (((1,), (1,)), ((), ()))


def _split_bf16(x):
    hi = x.astype(BF16)
    return hi, (x - hi.astype(F32)).astype(BF16)


def _dot3_nn(x, w_hi, w_lo):
    xh, xl = _split_bf16(x)
    dn = (((1,), (0,)), ((), ()))
    return (jax.lax.dot_general(xh, w_hi, dn, preferred_element_type=F32)
            + jax.lax.dot_general(xh, w_lo, dn, preferred_element_type=F32)
            + jax.lax.dot_general(xl, w_hi, dn, preferred_element_type=F32))


def _dot3_nt(x, w_hi, w_lo):
    xh, xl = _split_bf16(x)
    return (jax.lax.dot_general(xh, w_hi, _DN_NT, preferred_element_type=F32)
            + jax.lax.dot_general(xh, w_lo, _DN_NT, preferred_element_type=F32)
            + jax.lax.dot_general(xl, w_hi, _DN_NT, preferred_element_type=F32))


def _gat_pre_body(x_ref, w1_ref, as1_ref, ad1_ref, sum8_ref,
                  h_ref, als_ref, ald_ref):
    h = jnp.dot(x_ref[...], w1_ref[...], preferred_element_type=F32, precision=HI)
    h_ref[...] = h
    als = jnp.dot(h * as1_ref[...], sum8_ref[...], preferred_element_type=F32, precision=HI)
    ald = jnp.dot(h * ad1_ref[...], sum8_ref[...], preferred_element_type=F32, precision=HI)
    als_ref[...] = jnp.concatenate([als, als], axis=1)
    ald_ref[...] = jnp.concatenate([ald, ald], axis=1)


def _gat_mid_body(agg_ref, b1_ref, w2h_ref, w2l_ref, as2_ref, ad2_ref, sum8_ref,
                  h_ref, als_ref, ald_ref):
    o1 = agg_ref[...] + b1_ref[...]
    x1 = jnp.where(o1 > 0.0, o1, jnp.exp(jnp.minimum(o1, 0.0)) - 1.0)
    h = _dot3_nn(x1, w2h_ref[...], w2l_ref[...])
    h_ref[...] = h
    als = jnp.dot(h * as2_ref[...], sum8_ref[...], preferred_element_type=F32, precision=HI)
    ald = jnp.dot(h * ad2_ref[...], sum8_ref[...], preferred_element_type=F32, precision=HI)
    als_ref[...] = jnp.concatenate([als, als], axis=1)
    ald_ref[...] = jnp.concatenate([ald, ald], axis=1)


def _gat_post_body(agg_ref, avg_ref, b2_ref, out_ref):
    out_ref[...] = jnp.dot(agg_ref[...], avg_ref[...], preferred_element_type=F32, precision=HI) + b2_ref[...]


def _sc_agg_body(h_hbm, als_hbm, ald_hbm, idx_hbm, out_hbm,
                 h_v, als_v, ald_v, idx_v, coef_v, den_v, out_v,
                 s_sm, t_sm, a_sm):
    wid = jax.lax.axis_index("s") * 2 + jax.lax.axis_index("c")

    @pl.when(wid < G)
    def _():
        base = wid * GP
        pltpu.sync_copy(h_hbm.at[pl.ds(base, GP)], h_v)
        pltpu.sync_copy(als_hbm.at[pl.ds(base, GP)], als_v)
        pltpu.sync_copy(ald_hbm.at[pl.ds(base, GP)], ald_v)
        pltpu.sync_copy(idx_hbm, idx_v)

        for k in range(3):
            sv = idx_v[0, pl.ds(k * 16, 16)]
            tv = idx_v[1, pl.ds(k * 16, 16)]
            for j in range(16):
                e = k * 16 + j
                if e < E:
                    s_sm[e] = sv[j]
                    t_sm[e] = tv[j]

        def logit_body(e, m):
            s = s_sm[e]
            t = t_sm[e]
            le = als_v[s] + ald_v[t]
            le = jnp.maximum(le, 0.2 * le)
            coef_v[e] = le
            return jnp.maximum(m, le)
        m16 = jax.lax.fori_loop(0, E, logit_body, jnp.full((16,), -1e30, F32))

        def zden_body(v, c):
            den_v[v] = jnp.zeros((16,), F32)
            return c
        jax.lax.fori_loop(0, V, zden_body, 0)

        def exp_body(e, c):
            t = t_sm[e]
            ex = jnp.exp(coef_v[e] - m16)
            coef_v[e] = ex
            den_v[t] = den_v[t] + ex
            return c
        jax.lax.fori_loop(0, E, exp_body, 0)

        def norm_body(e, c):
            t = t_sm[e]
            a = coef_v[e] / (den_v[t] + 1e-16)
            for h in range(HEADS):
                a_sm[e * HEADS + h] = a[h]
            return c
        jax.lax.fori_loop(0, E, norm_body, 0)

        def zout_body(v, c):
            for ch in range(HID // 16):
                out_v[v, pl.ds(ch * 16, 16)] = jnp.zeros((16,), F32)
            return c
        jax.lax.fori_loop(0, GP, zout_body, 0)

        def agg_body(e, c):
            s = s_sm[e]
            t = t_sm[e]
            for h in range(HEADS):
                a = a_sm[e * HEADS + h]
                for ch in range(HC // 16):
                    sl = pl.ds(h * HC + ch * 16, 16)
                    out_v[t, sl] = out_v[t, sl] + a * h_v[s, sl]
            return c
        jax.lax.fori_loop(0, E, agg_body, 0)

        pltpu.sync_copy(out_v, out_hbm.at[pl.ds(base, GP)])


@functools.cache
def _sc_agg():
    return pl.kernel(
        _sc_agg_body,
        out_type=jax.ShapeDtypeStruct((NPV, HID), F32),
        mesh=plsc.VectorSubcoreMesh(core_axis_name="c", subcore_axis_name="s"),
        scratch_types=[
            pltpu.VMEM((GP, HID), F32),
            pltpu.VMEM((GP, 16), F32),
            pltpu.VMEM((GP, 16), F32),
            pltpu.VMEM((2, 48), jnp.int32),
            pltpu.VMEM((48, 16), F32),
            pltpu.VMEM((V, 16), F32),
            pltpu.VMEM((GP, HID), F32),
            pltpu.SMEM((48,), jnp.int32),
            pltpu.SMEM((48,), jnp.int32),
            pltpu.SMEM((E * HEADS + 16,), F32),
        ],
    )


def _edge_body(px_ref, py_ref, d0_ref, d1_ref, wa_ref, wb_ref, bfe_ref, out_ref):
    px, py = px_ref[...], py_ref[...]
    for r, d_ref in ((0, d0_ref), (1, d1_ref)):
        vx = jnp.dot(px, d_ref[...], preferred_element_type=F32, precision=HI)
        vy = jnp.dot(py, d_ref[...], preferred_element_type=F32, precision=HI)
        ln = jnp.sqrt(vx * vx + vy * vy)
        ang = jnp.arctan2(vy, vx)
        o = (jnp.dot(ln, wa_ref[...], preferred_element_type=F32, precision=HI)
             + jnp.dot(ang, wb_ref[...], preferred_element_type=F32, precision=HI)
             + bfe_ref[...])
        out_ref[:, r * HC:(r + 1) * HC] = o


def _proj_body(xe_ref, scv_ref, shv_ref, xg_ref, scg_ref, shg_ref, sel_ref,
               wteh_ref, wtel_ref, wtgh_ref, wtgl_ref, b_ref, out_ref):
    xg = xg_ref[...] * scg_ref[...] + shg_ref[...]
    g36 = _dot3_nt(xg, wtgh_ref[...], wtgl_ref[...])
    xe = xe_ref[...] * scv_ref[...] + shv_ref[...]
    out_ref[...] = (_dot3_nt(xe, wteh_ref[...], wtel_ref[...])
                    + jnp.dot(sel_ref[...], g36, preferred_element_type=F32, precision=HI)
                    + b_ref[0])


def _lstm_body(g_ref, whfh_ref, whfl_ref, whbh_ref, whbl_ref, watt_ref,
               wcls_ref, bcls_ref, att_ref, cls_ref, lo_ref):
    nb = 2 * B

    def cell(g):
        i = jax.nn.sigmoid(g[:, 0:HL])
        f = jax.nn.sigmoid(g[:, HL:2 * HL])
        gg = jnp.tanh(g[:, 2 * HL:3 * HL])
        o = jax.nn.sigmoid(g[:, 3 * HL:4 * HL])
        return i, f, gg, o

    hf = jnp.zeros((nb, HL), F32)
    cf = jnp.zeros((nb, HL), F32)
    hb = jnp.zeros((nb, HL), F32)
    cb = jnp.zeros((nb, HL), F32)
    for t in range(T):
        gf = g_ref[t, :, 0:4 * HL] + _dot3_nt(hf, whfh_ref[...], whfl_ref[...])
        i, f, gg, o = cell(gf)
        cf = f * cf + i * gg
        hf = o * jnp.tanh(cf)
        lo_ref[t, :, 0:HL] = hf
        tb = T - 1 - t
        gb = g_ref[tb, :, 4 * HL:8 * HL] + _dot3_nt(hb, whbh_ref[...], whbl_ref[...])
        i, f, gg, o = cell(gb)
        cb = f * cb + i * gg
        hb = o * jnp.tanh(cb)
        lo_ref[tb, :, HL:2 * HL] = hb

    scores = jnp.concatenate(
        [jnp.dot(lo_ref[t], watt_ref[...], preferred_element_type=F32, precision=HI)
         for t in range(T)], axis=1)
    m = jnp.max(scores, axis=1, keepdims=True)
    e = jnp.exp(scores - m)
    aw = e / jnp.sum(e, axis=1, keepdims=True)
    att = jnp.zeros((nb, 2 * HL), F32)
    for t in range(T):
        att = att + aw[:, t:t + 1] * lo_ref[t]
    att_ref[...] = att
    cls_ref[...] = jnp.dot(att, wcls_ref[...], preferred_element_type=F32, precision=HI) + bcls_ref[...]


def kernel(pose1, pose2, connections, W1, att_src1, att_dst1, b1, W2, att_src2,
           att_dst2, b2, Wfe, bfe, bn_gamma, bn_beta, bn_mean, bn_var, Wih_f,
           Whh_f, bih_f, bhh_f, Wih_b, Whh_b, bih_b, bhh_b, Watt, batt, Wcls,
           bcls):
    s_idx = connections[0].astype(jnp.int32)
    t_idx = connections[1].astype(jnp.int32)

    sum8 = jax.nn.one_hot(jnp.arange(HID, dtype=jnp.int32) // HC, HEADS, dtype=F32)
    avg8 = jax.nn.one_hot(jnp.arange(HID, dtype=jnp.int32) % HC, HC, dtype=F32) / HEADS
    idx48 = jnp.zeros((2, 48), jnp.int32).at[0, :E].set(s_idx).at[1, :E].set(t_idx)

    x24 = jnp.concatenate([pose1[0], pose2[0]], axis=0).reshape(G, V, 3)
    x_pad = jnp.zeros((G, GP, 3), F32).at[:, :V].set(x24).reshape(NPV, 3)
    h1, als1, ald1 = pl.pallas_call(
        _gat_pre_body,
        out_shape=(jax.ShapeDtypeStruct((NPV, HID), F32),
                   jax.ShapeDtypeStruct((NPV, 16), F32),
                   jax.ShapeDtypeStruct((NPV, 16), F32)),
    )(x_pad, W1, att_src1.reshape(1, HID), att_dst1.reshape(1, HID), sum8)
    agg1 = _sc_agg()(h1, als1, ald1, idx48)
    h2, als2, ald2 = pl.pallas_call(
        _gat_mid_body,
        out_shape=(jax.ShapeDtypeStruct((NPV, HID), F32),
                   jax.ShapeDtypeStruct((NPV, 16), F32),
                   jax.ShapeDtypeStruct((NPV, 16), F32)),
    )(agg1, b1.reshape(1, HID), *_split_bf16(W2), att_src2.reshape(1, HID),
      att_dst2.reshape(1, HID), sum8)
    agg2 = _sc_agg()(h2, als2, ald2, idx48)
    gat_nodes = pl.pallas_call(
        _gat_post_body,
        out_shape=jax.ShapeDtypeStruct((NPV, HC), F32),
    )(agg2, avg8, b2.reshape(1, HC))

    pall = jnp.concatenate([pose1, pose2], axis=0).transpose(1, 0, 2, 3)
    pall = pall.reshape(2 * B * T, V, 3)
    px, py = pall[:, :, 0], pall[:, :, 1]
    dmat = (jax.nn.one_hot(t_idx, V, dtype=F32) - jax.nn.one_hot(s_idx, V, dtype=F32)).T
    edge_out = pl.pallas_call(
        _edge_body,
        out_shape=jax.ShapeDtypeStruct((2 * B * T, 2 * HC), F32),
    )(px, py, dmat[:, :E // 2], dmat[:, E // 2:], Wfe[0::2], Wfe[1::2],
      bfe.reshape(1, HC))

    sc = bn_gamma / jnp.sqrt(bn_var + 1e-5)
    sh = bn_beta - bn_mean * sc
    gat2 = gat_nodes.reshape(G, GP, HC)[:, :V].reshape(G, V * HC)
    dead = jnp.tile(b2, V)
    xg36 = jnp.concatenate(
        [jnp.broadcast_to(dead, (T, V * HC)), gat2], axis=0)
    scg = jnp.tile(sc, 3).reshape(3 * T, 1)
    shg = jnp.tile(sh, 3).reshape(3 * T, 1)
    tcol = jnp.arange(2 * B * T, dtype=jnp.int32) // (2 * B)
    bcol = jnp.arange(2 * B * T, dtype=jnp.int32) % (2 * B)
    sel_idx = jnp.where(bcol == 0, 12 + tcol,
                        jnp.where(bcol == B, 12 + T + tcol, tcol))
    sel768 = jax.nn.one_hot(sel_idx, 3 * T, dtype=F32)

    scv = jnp.repeat(sc, 2 * B).reshape(2 * B * T, 1)
    shv = jnp.repeat(sh, 2 * B).reshape(2 * B * T, 1)
    wt = jnp.concatenate([Wih_f, Wih_b], axis=0)
    wteh, wtel = _split_bf16(wt[:, V * HC:])
    wtgh, wtgl = _split_bf16(wt[:, :V * HC])
    bias = jnp.concatenate([bih_f + bhh_f, bih_b + bhh_b]).reshape(8, 1, HL)
    nblk = 8
    proj = pl.pallas_call(
        _proj_body,
        grid=(nblk,),
        in_specs=[
            pl.BlockSpec((2 * B * T, 2 * HC), lambda i: (0, 0)),
            pl.BlockSpec((2 * B * T, 1), lambda i: (0, 0)),
            pl.BlockSpec((2 * B * T, 1), lambda i: (0, 0)),
            pl.BlockSpec((3 * T, V * HC), lambda i: (0, 0)),
            pl.BlockSpec((3 * T, 1), lambda i: (0, 0)),
            pl.BlockSpec((3 * T, 1), lambda i: (0, 0)),
            pl.BlockSpec((2 * B * T, 3 * T), lambda i: (0, 0)),
            pl.BlockSpec((HL, 2 * HC), lambda i: (i, 0)),
            pl.BlockSpec((HL, 2 * HC), lambda i: (i, 0)),
            pl.BlockSpec((HL, V * HC), lambda i: (i, 0)),
            pl.BlockSpec((HL, V * HC), lambda i: (i, 0)),
            pl.BlockSpec((1, 1, HL), lambda i: (i, 0, 0)),
        ],
        out_specs=pl.BlockSpec((2 * B * T, HL), lambda i: (0, i)),
        out_shape=jax.ShapeDtypeStruct((2 * B * T, 8 * HL), F32),
    )(edge_out, scv, shv, xg36, scg, shg, sel768, wteh, wtel, wtgh, wtgl, bias)

    whfh, whfl = _split_bf16(Whh_f)
    whbh, whbl = _split_bf16(Whh_b)
    att, cls = pl.pallas_call(
        _lstm_body,
        out_shape=(jax.ShapeDtypeStruct((2 * B, 2 * HL), F32),
                   jax.ShapeDtypeStruct((2 * B, NCLS), F32)),
        scratch_shapes=[pltpu.VMEM((T, 2 * B, 2 * HL), F32)],
    )(proj.reshape(T, 2 * B, 8 * HL), whfh, whfl, whbh, whbl, Watt, Wcls,
      bcls.reshape(1, NCLS))
    return att, cls


# speedup vs baseline: 1.0419x; 1.0419x over previous
import functools

import jax
import jax.numpy as jnp
from jax.experimental import pallas as pl
from jax.experimental.pallas import tpu as pltpu
from jax.experimental.pallas import tpu_sc as plsc

B, T, V, E = 32, 12, 17, 38
HEADS, HC, HL, NCLS = 8, 128, 512, 500
G = 2 * T
NGV = G * V
NGE = G * E
HID = HEADS * HC
D = HC * (V + 2)
GP = 24
NPV = G * GP
F32 = jnp.float32
HI = jax.lax.Precision.HIGHEST
BF16 = jnp.bfloat16
_DN_NT = (((1,), (1,)), ((), ()))


def _split_bf16(x):
    hi = x.astype(BF16)
    return hi, (x - hi.astype(F32)).astype(BF16)


def _dot3_nn(x, w_hi, w_lo):
    xh, xl = _split_bf16(x)
    dn = (((1,), (0,)), ((), ()))
    return (jax.lax.dot_general(xh, w_hi, dn, preferred_element_type=F32)
            + jax.lax.dot_general(xh, w_lo, dn, preferred_element_type=F32)
            + jax.lax.dot_general(xl, w_hi, dn, preferred_element_type=F32))


def _dot3_nt(x, w_hi, w_lo):
    xh, xl = _split_bf16(x)
    return (jax.lax.dot_general(xh, w_hi, _DN_NT, preferred_element_type=F32)
            + jax.lax.dot_general(xh, w_lo, _DN_NT, preferred_element_type=F32)
            + jax.lax.dot_general(xl, w_hi, _DN_NT, preferred_element_type=F32))


def _gat_pre_body(x_ref, w1_ref, as1_ref, ad1_ref, sum8_ref,
                  h_ref, als_ref, ald_ref):
    h = jnp.dot(x_ref[...], w1_ref[...], preferred_element_type=F32, precision=HI)
    h_ref[...] = h
    als = jnp.dot(h * as1_ref[...], sum8_ref[...], preferred_element_type=F32, precision=HI)
    ald = jnp.dot(h * ad1_ref[...], sum8_ref[...], preferred_element_type=F32, precision=HI)
    als_ref[...] = jnp.concatenate([als, als], axis=1)
    ald_ref[...] = jnp.concatenate([ald, ald], axis=1)


def _gat_mid_body(agg_ref, b1_ref, w2_ref, as2_ref, ad2_ref, sum8_ref,
                  h_ref, als_ref, ald_ref):
    o1 = agg_ref[...] + b1_ref[...]
    x1 = jnp.where(o1 > 0.0, o1, jnp.exp(jnp.minimum(o1, 0.0)) - 1.0)
    h = jnp.dot(x1, w2_ref[...], preferred_element_type=F32, precision=HI)
    h_ref[...] = h
    als = jnp.dot(h * as2_ref[...], sum8_ref[...], preferred_element_type=F32, precision=HI)
    ald = jnp.dot(h * ad2_ref[...], sum8_ref[...], preferred_element_type=F32, precision=HI)
    als_ref[...] = jnp.concatenate([als, als], axis=1)
    ald_ref[...] = jnp.concatenate([ald, ald], axis=1)


def _gat_post_body(agg_ref, avg_ref, b2_ref, out_ref):
    out_ref[...] = jnp.dot(agg_ref[...], avg_ref[...], preferred_element_type=F32, precision=HI) + b2_ref[...]


def _sc_agg_body(h_hbm, als_hbm, ald_hbm, idx_hbm, out_hbm,
                 h_v, als_v, ald_v, idx_v, coef_v, den_v, out_v,
                 s_sm, t_sm, a_sm):
    wid = jax.lax.axis_index("s") * 2 + jax.lax.axis_index("c")

    @pl.when(wid < G)
    def _():
        base = wid * GP
        pltpu.sync_copy(h_hbm.at[pl.ds(base, GP)], h_v)
        pltpu.sync_copy(als_hbm.at[pl.ds(base, GP)], als_v)
        pltpu.sync_copy(ald_hbm.at[pl.ds(base, GP)], ald_v)
        pltpu.sync_copy(idx_hbm, idx_v)

        for k in range(3):
            sv = idx_v[0, pl.ds(k * 16, 16)]
            tv = idx_v[1, pl.ds(k * 16, 16)]
            for j in range(16):
                e = k * 16 + j
                if e < E:
                    s_sm[e] = sv[j]
                    t_sm[e] = tv[j]

        def logit_body(e, m):
            s = s_sm[e]
            t = t_sm[e]
            le = als_v[s] + ald_v[t]
            le = jnp.maximum(le, 0.2 * le)
            coef_v[e] = le
            return jnp.maximum(m, le)
        m16 = jax.lax.fori_loop(0, E, logit_body, jnp.full((16,), -1e30, F32))

        def zden_body(v, c):
            den_v[v] = jnp.zeros((16,), F32)
            return c
        jax.lax.fori_loop(0, V, zden_body, 0)

        def exp_body(e, c):
            t = t_sm[e]
            ex = jnp.exp(coef_v[e] - m16)
            coef_v[e] = ex
            den_v[t] = den_v[t] + ex
            return c
        jax.lax.fori_loop(0, E, exp_body, 0)

        def norm_body(e, c):
            t = t_sm[e]
            a = coef_v[e] / (den_v[t] + 1e-16)
            for h in range(HEADS):
                a_sm[e * HEADS + h] = a[h]
            return c
        jax.lax.fori_loop(0, E, norm_body, 0)

        def zout_body(v, c):
            for ch in range(HID // 16):
                out_v[v, pl.ds(ch * 16, 16)] = jnp.zeros((16,), F32)
            return c
        jax.lax.fori_loop(0, GP, zout_body, 0)

        def agg_body(e, c):
            s = s_sm[e]
            t = t_sm[e]
            for h in range(HEADS):
                a = a_sm[e * HEADS + h]
                for ch in range(HC // 16):
                    sl = pl.ds(h * HC + ch * 16, 16)
                    out_v[t, sl] = out_v[t, sl] + a * h_v[s, sl]
            return c
        jax.lax.fori_loop(0, E, agg_body, 0)

        pltpu.sync_copy(out_v, out_hbm.at[pl.ds(base, GP)])


@functools.cache
def _sc_agg():
    return pl.kernel(
        _sc_agg_body,
        out_type=jax.ShapeDtypeStruct((NPV, HID), F32),
        mesh=plsc.VectorSubcoreMesh(core_axis_name="c", subcore_axis_name="s"),
        scratch_types=[
            pltpu.VMEM((GP, HID), F32),
            pltpu.VMEM((GP, 16), F32),
            pltpu.VMEM((GP, 16), F32),
            pltpu.VMEM((2, 48), jnp.int32),
            pltpu.VMEM((48, 16), F32),
            pltpu.VMEM((V, 16), F32),
            pltpu.VMEM((GP, HID), F32),
            pltpu.SMEM((48,), jnp.int32),
            pltpu.SMEM((48,), jnp.int32),
            pltpu.SMEM((E * HEADS + 16,), F32),
        ],
    )


def _edge_body(px_ref, py_ref, d0_ref, d1_ref, wa_ref, wb_ref, bfe_ref, out_ref):
    px, py = px_ref[...], py_ref[...]
    for r, d_ref in ((0, d0_ref), (1, d1_ref)):
        vx = jnp.dot(px, d_ref[...], preferred_element_type=F32, precision=HI)
        vy = jnp.dot(py, d_ref[...], preferred_element_type=F32, precision=HI)
        ln = jnp.sqrt(vx * vx + vy * vy)
        ang = jnp.arctan2(vy, vx)
        o = (jnp.dot(ln, wa_ref[...], preferred_element_type=F32, precision=HI)
             + jnp.dot(ang, wb_ref[...], preferred_element_type=F32, precision=HI)
             + bfe_ref[...])
        out_ref[:, r * HC:(r + 1) * HC] = o


def _proj_body(xe_ref, scv_ref, shv_ref, xg_ref, scg_ref, shg_ref, sel_ref,
               wte_ref, wtg_ref, b_ref, out_ref):
    xg = xg_ref[...] * scg_ref[...] + shg_ref[...]
    g36 = jax.lax.dot_general(xg, wtg_ref[...], _DN_NT, preferred_element_type=F32, precision=HI)
    xe = xe_ref[...] * scv_ref[...] + shv_ref[...]
    out_ref[...] = (jax.lax.dot_general(xe, wte_ref[...], _DN_NT, preferred_element_type=F32, precision=HI)
                    + jnp.dot(sel_ref[...], g36, preferred_element_type=F32, precision=HI)
                    + b_ref[0])


def _lstm_body(g_ref, whfh_ref, whfl_ref, whbh_ref, whbl_ref, watt_ref,
               wcls_ref, bcls_ref, att_ref, cls_ref, lo_ref):
    nb = 2 * B

    def cell(g):
        i = jax.nn.sigmoid(g[:, 0:HL])
        f = jax.nn.sigmoid(g[:, HL:2 * HL])
        gg = jnp.tanh(g[:, 2 * HL:3 * HL])
        o = jax.nn.sigmoid(g[:, 3 * HL:4 * HL])
        return i, f, gg, o

    hf = jnp.zeros((nb, HL), F32)
    cf = jnp.zeros((nb, HL), F32)
    hb = jnp.zeros((nb, HL), F32)
    cb = jnp.zeros((nb, HL), F32)
    for t in range(T):
        gf = g_ref[t, :, 0:4 * HL] + _dot3_nt(hf, whfh_ref[...], whfl_ref[...])
        i, f, gg, o = cell(gf)
        cf = f * cf + i * gg
        hf = o * jnp.tanh(cf)
        lo_ref[t, :, 0:HL] = hf
        tb = T - 1 - t
        gb = g_ref[tb, :, 4 * HL:8 * HL] + _dot3_nt(hb, whbh_ref[...], whbl_ref[...])
        i, f, gg, o = cell(gb)
        cb = f * cb + i * gg
        hb = o * jnp.tanh(cb)
        lo_ref[tb, :, HL:2 * HL] = hb

    scores = jnp.concatenate(
        [jnp.dot(lo_ref[t], watt_ref[...], preferred_element_type=F32, precision=HI)
         for t in range(T)], axis=1)
    m = jnp.max(scores, axis=1, keepdims=True)
    e = jnp.exp(scores - m)
    aw = e / jnp.sum(e, axis=1, keepdims=True)
    att = jnp.zeros((nb, 2 * HL), F32)
    for t in range(T):
        att = att + aw[:, t:t + 1] * lo_ref[t]
    att_ref[...] = att
    cls_ref[...] = jnp.dot(att, wcls_ref[...], preferred_element_type=F32, precision=HI) + bcls_ref[...]


def kernel(pose1, pose2, connections, W1, att_src1, att_dst1, b1, W2, att_src2,
           att_dst2, b2, Wfe, bfe, bn_gamma, bn_beta, bn_mean, bn_var, Wih_f,
           Whh_f, bih_f, bhh_f, Wih_b, Whh_b, bih_b, bhh_b, Watt, batt, Wcls,
           bcls):
    s_idx = connections[0].astype(jnp.int32)
    t_idx = connections[1].astype(jnp.int32)

    sum8 = jax.nn.one_hot(jnp.arange(HID, dtype=jnp.int32) // HC, HEADS, dtype=F32)
    avg8 = jax.nn.one_hot(jnp.arange(HID, dtype=jnp.int32) % HC, HC, dtype=F32) / HEADS
    idx48 = jnp.zeros((2, 48), jnp.int32).at[0, :E].set(s_idx).at[1, :E].set(t_idx)

    x24 = jnp.concatenate([pose1[0], pose2[0]], axis=0).reshape(G, V, 3)
    x_pad = jnp.zeros((G, GP, 3), F32).at[:, :V].set(x24).reshape(NPV, 3)
    h1, als1, ald1 = pl.pallas_call(
        _gat_pre_body,
        out_shape=(jax.ShapeDtypeStruct((NPV, HID), F32),
                   jax.ShapeDtypeStruct((NPV, 16), F32),
                   jax.ShapeDtypeStruct((NPV, 16), F32)),
    )(x_pad, W1, att_src1.reshape(1, HID), att_dst1.reshape(1, HID), sum8)
    agg1 = _sc_agg()(h1, als1, ald1, idx48)
    h2, als2, ald2 = pl.pallas_call(
        _gat_mid_body,
        out_shape=(jax.ShapeDtypeStruct((NPV, HID), F32),
                   jax.ShapeDtypeStruct((NPV, 16), F32),
                   jax.ShapeDtypeStruct((NPV, 16), F32)),
    )(agg1, b1.reshape(1, HID), W2, att_src2.reshape(1, HID),
      att_dst2.reshape(1, HID), sum8)
    agg2 = _sc_agg()(h2, als2, ald2, idx48)
    gat_nodes = pl.pallas_call(
        _gat_post_body,
        out_shape=jax.ShapeDtypeStruct((NPV, HC), F32),
    )(agg2, avg8, b2.reshape(1, HC))

    pall = jnp.concatenate([pose1, pose2], axis=0).transpose(1, 0, 2, 3)
    pall = pall.reshape(2 * B * T, V, 3)
    px, py = pall[:, :, 0], pall[:, :, 1]
    dmat = (jax.nn.one_hot(t_idx, V, dtype=F32) - jax.nn.one_hot(s_idx, V, dtype=F32)).T
    edge_out = pl.pallas_call(
        _edge_body,
        out_shape=jax.ShapeDtypeStruct((2 * B * T, 2 * HC), F32),
    )(px, py, dmat[:, :E // 2], dmat[:, E // 2:], Wfe[0::2], Wfe[1::2],
      bfe.reshape(1, HC))

    sc = bn_gamma / jnp.sqrt(bn_var + 1e-5)
    sh = bn_beta - bn_mean * sc
    gat2 = gat_nodes.reshape(G, GP, HC)[:, :V].reshape(G, V * HC)
    dead = jnp.tile(b2, V)
    xg36 = jnp.concatenate(
        [jnp.broadcast_to(dead, (T, V * HC)), gat2], axis=0)
    scg = jnp.tile(sc, 3).reshape(3 * T, 1)
    shg = jnp.tile(sh, 3).reshape(3 * T, 1)
    tcol = jnp.arange(2 * B * T, dtype=jnp.int32) // (2 * B)
    bcol = jnp.arange(2 * B * T, dtype=jnp.int32) % (2 * B)
    sel_idx = jnp.where(bcol == 0, 12 + tcol,
                        jnp.where(bcol == B, 12 + T + tcol, tcol))
    sel768 = jax.nn.one_hot(sel_idx, 3 * T, dtype=F32)

    scv = jnp.repeat(sc, 2 * B).reshape(2 * B * T, 1)
    shv = jnp.repeat(sh, 2 * B).reshape(2 * B * T, 1)
    wt = jnp.concatenate([Wih_f, Wih_b], axis=0)
    wte = wt[:, V * HC:]
    wtg = wt[:, :V * HC]
    bias = jnp.concatenate([bih_f + bhh_f, bih_b + bhh_b]).reshape(8, 1, HL)
    nblk = 8
    proj = pl.pallas_call(
        _proj_body,
        grid=(nblk,),
        in_specs=[
            pl.BlockSpec((2 * B * T, 2 * HC), lambda i: (0, 0)),
            pl.BlockSpec((2 * B * T, 1), lambda i: (0, 0)),
            pl.BlockSpec((2 * B * T, 1), lambda i: (0, 0)),
            pl.BlockSpec((3 * T, V * HC), lambda i: (0, 0)),
            pl.BlockSpec((3 * T, 1), lambda i: (0, 0)),
            pl.BlockSpec((3 * T, 1), lambda i: (0, 0)),
            pl.BlockSpec((2 * B * T, 3 * T), lambda i: (0, 0)),
            pl.BlockSpec((HL, 2 * HC), lambda i: (i, 0)),
            pl.BlockSpec((HL, V * HC), lambda i: (i, 0)),
            pl.BlockSpec((1, 1, HL), lambda i: (i, 0, 0)),
        ],
        out_specs=pl.BlockSpec((2 * B * T, HL), lambda i: (0, i)),
        out_shape=jax.ShapeDtypeStruct((2 * B * T, 8 * HL), F32),
    )(edge_out, scv, shv, xg36, scg, shg, sel768, wte, wtg, bias)

    whfh, whfl = _split_bf16(Whh_f)
    whbh, whbl = _split_bf16(Whh_b)
    att, cls = pl.pallas_call(
        _lstm_body,
        out_shape=(jax.ShapeDtypeStruct((2 * B, 2 * HL), F32),
                   jax.ShapeDtypeStruct((2 * B, NCLS), F32)),
        scratch_shapes=[pltpu.VMEM((T, 2 * B, 2 * HL), F32)],
    )(proj.reshape(T, 2 * B, 8 * HL), whfh, whfl, whbh, whbl, Watt, Wcls,
      bcls.reshape(1, NCLS))
    return att, cls


# speedup vs baseline: 1.1970x; 1.1488x over previous
import functools

import jax
import jax.numpy as jnp
from jax.experimental import pallas as pl
from jax.experimental.pallas import tpu as pltpu
from jax.experimental.pallas import tpu_sc as plsc

B, T, V, E = 32, 12, 17, 38
HEADS, HC, HL, NCLS = 8, 128, 512, 500
G = 2 * T
NGV = G * V
NGE = G * E
HID = HEADS * HC
D = HC * (V + 2)
GP = 24
NPV = G * GP
F32 = jnp.float32
HI = jax.lax.Precision.HIGHEST
BF16 = jnp.bfloat16
_DN_NT = (((1,), (1,)), ((), ()))


def _split_bf16(x):
    hi = x.astype(BF16)
    return hi, (x - hi.astype(F32)).astype(BF16)


def _dot3_nn(x, w_hi, w_lo):
    xh, xl = _split_bf16(x)
    dn = (((1,), (0,)), ((), ()))
    return (jax.lax.dot_general(xh, w_hi, dn, preferred_element_type=F32)
            + jax.lax.dot_general(xh, w_lo, dn, preferred_element_type=F32)
            + jax.lax.dot_general(xl, w_hi, dn, preferred_element_type=F32))


def _dot3_nt(x, w_hi, w_lo):
    xh, xl = _split_bf16(x)
    return (jax.lax.dot_general(xh, w_hi, _DN_NT, preferred_element_type=F32)
            + jax.lax.dot_general(xh, w_lo, _DN_NT, preferred_element_type=F32)
            + jax.lax.dot_general(xl, w_hi, _DN_NT, preferred_element_type=F32))


def _gat_pre_body(x_ref, w1_ref, as1_ref, ad1_ref, sum8_ref,
                  h_ref, als_ref, ald_ref):
    h = jnp.dot(x_ref[...], w1_ref[...], preferred_element_type=F32, precision=HI)
    h_ref[...] = h
    als = jnp.dot(h * as1_ref[...], sum8_ref[...], preferred_element_type=F32, precision=HI)
    ald = jnp.dot(h * ad1_ref[...], sum8_ref[...], preferred_element_type=F32, precision=HI)
    als_ref[...] = jnp.concatenate([als, als], axis=1)
    ald_ref[...] = jnp.concatenate([ald, ald], axis=1)


def _gat_mid_body(agg_ref, b1_ref, w2_ref, as2_ref, ad2_ref, sum8_ref,
                  h_ref, als_ref, ald_ref):
    o1 = agg_ref[...] + b1_ref[...]
    x1 = jnp.where(o1 > 0.0, o1, jnp.exp(jnp.minimum(o1, 0.0)) - 1.0)
    h = jnp.dot(x1, w2_ref[...], preferred_element_type=F32, precision=HI)
    h_ref[...] = h
    als = jnp.dot(h * as2_ref[...], sum8_ref[...], preferred_element_type=F32, precision=HI)
    ald = jnp.dot(h * ad2_ref[...], sum8_ref[...], preferred_element_type=F32, precision=HI)
    als_ref[...] = jnp.concatenate([als, als], axis=1)
    ald_ref[...] = jnp.concatenate([ald, ald], axis=1)


def _gat_post_body(agg_ref, avg_ref, b2_ref, out_ref):
    out_ref[...] = jnp.dot(agg_ref[...], avg_ref[...], preferred_element_type=F32, precision=HI) + b2_ref[...]


def _sc_agg_body(h_hbm, als_hbm, ald_hbm, idx_hbm, out_hbm,
                 h_v, als_v, ald_v, idx_v, coef_v, den_v, out_v,
                 s_sm, t_sm, a_sm):
    wid = jax.lax.axis_index("s") * 2 + jax.lax.axis_index("c")

    @pl.when(wid < G)
    def _():
        base = wid * GP
        pltpu.sync_copy(h_hbm.at[pl.ds(base, GP)], h_v)
        pltpu.sync_copy(als_hbm.at[pl.ds(base, GP)], als_v)
        pltpu.sync_copy(ald_hbm.at[pl.ds(base, GP)], ald_v)
        pltpu.sync_copy(idx_hbm, idx_v)

        for k in range(3):
            sv = idx_v[0, pl.ds(k * 16, 16)]
            tv = idx_v[1, pl.ds(k * 16, 16)]
            for j in range(16):
                e = k * 16 + j
                if e < E:
                    s_sm[e] = sv[j]
                    t_sm[e] = tv[j]

        def logit_body(e, m):
            s = s_sm[e]
            t = t_sm[e]
            le = als_v[s] + ald_v[t]
            le = jnp.maximum(le, 0.2 * le)
            coef_v[e] = le
            return jnp.maximum(m, le)
        m16 = jax.lax.fori_loop(0, E, logit_body, jnp.full((16,), -1e30, F32))

        def zden_body(v, c):
            den_v[v] = jnp.zeros((16,), F32)
            return c
        jax.lax.fori_loop(0, V, zden_body, 0)

        def exp_body(e, c):
            t = t_sm[e]
            ex = jnp.exp(coef_v[e] - m16)
            coef_v[e] = ex
            den_v[t] = den_v[t] + ex
            return c
        jax.lax.fori_loop(0, E, exp_body, 0)

        def norm_body(e, c):
            t = t_sm[e]
            a = coef_v[e] / (den_v[t] + 1e-16)
            for h in range(HEADS):
                a_sm[e * HEADS + h] = a[h]
            return c
        jax.lax.fori_loop(0, E, norm_body, 0)

        def zout_body(v, c):
            for ch in range(HID // 16):
                out_v[v, pl.ds(ch * 16, 16)] = jnp.zeros((16,), F32)
            return c
        jax.lax.fori_loop(0, GP, zout_body, 0)

        def agg_body(e, c):
            s = s_sm[e]
            t = t_sm[e]
            for h in range(HEADS):
                a = a_sm[e * HEADS + h]
                for ch in range(HC // 16):
                    sl = pl.ds(h * HC + ch * 16, 16)
                    out_v[t, sl] = out_v[t, sl] + a * h_v[s, sl]
            return c
        jax.lax.fori_loop(0, E, agg_body, 0)

        pltpu.sync_copy(out_v, out_hbm.at[pl.ds(base, GP)])


@functools.cache
def _sc_agg():
    return pl.kernel(
        _sc_agg_body,
        out_type=jax.ShapeDtypeStruct((NPV, HID), F32),
        mesh=plsc.VectorSubcoreMesh(core_axis_name="c", subcore_axis_name="s"),
        scratch_types=[
            pltpu.VMEM((GP, HID), F32),
            pltpu.VMEM((GP, 16), F32),
            pltpu.VMEM((GP, 16), F32),
            pltpu.VMEM((2, 48), jnp.int32),
            pltpu.VMEM((48, 16), F32),
            pltpu.VMEM((V, 16), F32),
            pltpu.VMEM((GP, HID), F32),
            pltpu.SMEM((48,), jnp.int32),
            pltpu.SMEM((48,), jnp.int32),
            pltpu.SMEM((E * HEADS + 16,), F32),
        ],
    )


def _edge_body(px_ref, py_ref, d0_ref, d1_ref, wa_ref, wb_ref, bfe_ref, out_ref):
    px, py = px_ref[...], py_ref[...]
    for r, d_ref in ((0, d0_ref), (1, d1_ref)):
        vx = jnp.dot(px, d_ref[...], preferred_element_type=F32, precision=HI)
        vy = jnp.dot(py, d_ref[...], preferred_element_type=F32, precision=HI)
        ln = jnp.sqrt(vx * vx + vy * vy)
        ang = jnp.arctan2(vy, vx)
        o = (jnp.dot(ln, wa_ref[...], preferred_element_type=F32, precision=HI)
             + jnp.dot(ang, wb_ref[...], preferred_element_type=F32, precision=HI)
             + bfe_ref[...])
        out_ref[:, r * HC:(r + 1) * HC] = o


def _proj_body(xe_ref, scv_ref, shv_ref, xg_ref, scg_ref, shg_ref, sel_ref,
               wte_ref, wtg_ref, b_ref, out_ref):
    xg = xg_ref[...] * scg_ref[...] + shg_ref[...]
    g36 = jax.lax.dot_general(xg, wtg_ref[...], _DN_NT, preferred_element_type=F32)
    xe = xe_ref[...] * scv_ref[...] + shv_ref[...]
    out_ref[...] = (jax.lax.dot_general(xe, wte_ref[...], _DN_NT, preferred_element_type=F32)
                    + jnp.dot(sel_ref[...], g36, preferred_element_type=F32, precision=HI)
                    + b_ref[0])


def _lstm_body(g_ref, whfh_ref, whfl_ref, whbh_ref, whbl_ref, watt_ref,
               wcls_ref, bcls_ref, att_ref, cls_ref, lo_ref):
    nb = 2 * B

    def cell(g):
        i = jax.nn.sigmoid(g[:, 0:HL])
        f = jax.nn.sigmoid(g[:, HL:2 * HL])
        gg = jnp.tanh(g[:, 2 * HL:3 * HL])
        o = jax.nn.sigmoid(g[:, 3 * HL:4 * HL])
        return i, f, gg, o

    hf = jnp.zeros((nb, HL), F32)
    cf = jnp.zeros((nb, HL), F32)
    hb = jnp.zeros((nb, HL), F32)
    cb = jnp.zeros((nb, HL), F32)
    for t in range(T):
        gf = g_ref[t, :, 0:4 * HL] + _dot3_nt(hf, whfh_ref[...], whfl_ref[...])
        i, f, gg, o = cell(gf)
        cf = f * cf + i * gg
        hf = o * jnp.tanh(cf)
        lo_ref[t, :, 0:HL] = hf
        tb = T - 1 - t
        gb = g_ref[tb, :, 4 * HL:8 * HL] + _dot3_nt(hb, whbh_ref[...], whbl_ref[...])
        i, f, gg, o = cell(gb)
        cb = f * cb + i * gg
        hb = o * jnp.tanh(cb)
        lo_ref[tb, :, HL:2 * HL] = hb

    scores = jnp.concatenate(
        [jnp.dot(lo_ref[t], watt_ref[...], preferred_element_type=F32, precision=HI)
         for t in range(T)], axis=1)
    m = jnp.max(scores, axis=1, keepdims=True)
    e = jnp.exp(scores - m)
    aw = e / jnp.sum(e, axis=1, keepdims=True)
    att = jnp.zeros((nb, 2 * HL), F32)
    for t in range(T):
        att = att + aw[:, t:t + 1] * lo_ref[t]
    att_ref[...] = att
    cls_ref[...] = jnp.dot(att, wcls_ref[...], preferred_element_type=F32, precision=HI) + bcls_ref[...]


def kernel(pose1, pose2, connections, W1, att_src1, att_dst1, b1, W2, att_src2,
           att_dst2, b2, Wfe, bfe, bn_gamma, bn_beta, bn_mean, bn_var, Wih_f,
           Whh_f, bih_f, bhh_f, Wih_b, Whh_b, bih_b, bhh_b, Watt, batt, Wcls,
           bcls):
    s_idx = connections[0].astype(jnp.int32)
    t_idx = connections[1].astype(jnp.int32)

    sum8 = jax.nn.one_hot(jnp.arange(HID, dtype=jnp.int32) // HC, HEADS, dtype=F32)
    avg8 = jax.nn.one_hot(jnp.arange(HID, dtype=jnp.int32) % HC, HC, dtype=F32) / HEADS
    idx48 = jnp.zeros((2, 48), jnp.int32).at[0, :E].set(s_idx).at[1, :E].set(t_idx)

    x24 = jnp.concatenate([pose1[0], pose2[0]], axis=0).reshape(G, V, 3)
    x_pad = jnp.zeros((G, GP, 3), F32).at[:, :V].set(x24).reshape(NPV, 3)
    h1, als1, ald1 = pl.pallas_call(
        _gat_pre_body,
        out_shape=(jax.ShapeDtypeStruct((NPV, HID), F32),
                   jax.ShapeDtypeStruct((NPV, 16), F32),
                   jax.ShapeDtypeStruct((NPV, 16), F32)),
    )(x_pad, W1, att_src1.reshape(1, HID), att_dst1.reshape(1, HID), sum8)
    agg1 = _sc_agg()(h1, als1, ald1, idx48)
    h2, als2, ald2 = pl.pallas_call(
        _gat_mid_body,
        out_shape=(jax.ShapeDtypeStruct((NPV, HID), F32),
                   jax.ShapeDtypeStruct((NPV, 16), F32),
                   jax.ShapeDtypeStruct((NPV, 16), F32)),
    )(agg1, b1.reshape(1, HID), W2, att_src2.reshape(1, HID),
      att_dst2.reshape(1, HID), sum8)
    agg2 = _sc_agg()(h2, als2, ald2, idx48)
    gat_nodes = pl.pallas_call(
        _gat_post_body,
        out_shape=jax.ShapeDtypeStruct((NPV, HC), F32),
    )(agg2, avg8, b2.reshape(1, HC))

    pall = jnp.concatenate([pose1, pose2], axis=0).transpose(1, 0, 2, 3)
    pall = pall.reshape(2 * B * T, V, 3)
    px, py = pall[:, :, 0], pall[:, :, 1]
    dmat = (jax.nn.one_hot(t_idx, V, dtype=F32) - jax.nn.one_hot(s_idx, V, dtype=F32)).T
    edge_out = pl.pallas_call(
        _edge_body,
        out_shape=jax.ShapeDtypeStruct((2 * B * T, 2 * HC), F32),
    )(px, py, dmat[:, :E // 2], dmat[:, E // 2:], Wfe[0::2], Wfe[1::2],
      bfe.reshape(1, HC))

    sc = bn_gamma / jnp.sqrt(bn_var + 1e-5)
    sh = bn_beta - bn_mean * sc
    gat2 = gat_nodes.reshape(G, GP, HC)[:, :V].reshape(G, V * HC)
    dead = jnp.tile(b2, V)
    xg36 = jnp.concatenate(
        [jnp.broadcast_to(dead, (T, V * HC)), gat2], axis=0)
    scg = jnp.tile(sc, 3).reshape(3 * T, 1)
    shg = jnp.tile(sh, 3).reshape(3 * T, 1)
    tcol = jnp.arange(2 * B * T, dtype=jnp.int32) // (2 * B)
    bcol = jnp.arange(2 * B * T, dtype=jnp.int32) % (2 * B)
    sel_idx = jnp.where(bcol == 0, 12 + tcol,
                        jnp.where(bcol == B, 12 + T + tcol, tcol))
    sel768 = jax.nn.one_hot(sel_idx, 3 * T, dtype=F32)

    scv = jnp.repeat(sc, 2 * B).reshape(2 * B * T, 1)
    shv = jnp.repeat(sh, 2 * B).reshape(2 * B * T, 1)
    wt = jnp.concatenate([Wih_f, Wih_b], axis=0)
    wte = wt[:, V * HC:]
    wtg = wt[:, :V * HC]
    bias = jnp.concatenate([bih_f + bhh_f, bih_b + bhh_b]).reshape(8, 1, HL)
    nblk = 8
    proj = pl.pallas_call(
        _proj_body,
        grid=(nblk,),
        in_specs=[
            pl.BlockSpec((2 * B * T, 2 * HC), lambda i: (0, 0)),
            pl.BlockSpec((2 * B * T, 1), lambda i: (0, 0)),
            pl.BlockSpec((2 * B * T, 1), lambda i: (0, 0)),
            pl.BlockSpec((3 * T, V * HC), lambda i: (0, 0)),
            pl.BlockSpec((3 * T, 1), lambda i: (0, 0)),
            pl.BlockSpec((3 * T, 1), lambda i: (0, 0)),
            pl.BlockSpec((2 * B * T, 3 * T), lambda i: (0, 0)),
            pl.BlockSpec((HL, 2 * HC), lambda i: (i, 0)),
            pl.BlockSpec((HL, V * HC), lambda i: (i, 0)),
            pl.BlockSpec((1, 1, HL), lambda i: (i, 0, 0)),
        ],
        out_specs=pl.BlockSpec((2 * B * T, HL), lambda i: (0, i)),
        out_shape=jax.ShapeDtypeStruct((2 * B * T, 8 * HL), F32),
    )(edge_out, scv, shv, xg36, scg, shg, sel768, wte, wtg, bias)

    whfh, whfl = _split_bf16(Whh_f)
    whbh, whbl = _split_bf16(Whh_b)
    att, cls = pl.pallas_call(
        _lstm_body,
        out_shape=(jax.ShapeDtypeStruct((2 * B, 2 * HL), F32),
                   jax.ShapeDtypeStruct((2 * B, NCLS), F32)),
        scratch_shapes=[pltpu.VMEM((T, 2 * B, 2 * HL), F32)],
    )(proj.reshape(T, 2 * B, 8 * HL), whfh, whfl, whbh, whbl, Watt, Wcls,
      bcls.reshape(1, NCLS))
    return att, cls


# speedup vs baseline: 1.3600x; 1.1362x over previous
import functools

import jax
import jax.numpy as jnp
from jax.experimental import pallas as pl
from jax.experimental.pallas import tpu as pltpu
from jax.experimental.pallas import tpu_sc as plsc

B, T, V, E = 32, 12, 17, 38
HEADS, HC, HL, NCLS = 8, 128, 512, 500
G = 2 * T
NGV = G * V
NGE = G * E
HID = HEADS * HC
D = HC * (V + 2)
GP = 24
NPV = G * GP
F32 = jnp.float32
HI = jax.lax.Precision.HIGHEST
BF16 = jnp.bfloat16
_DN_NT = (((1,), (1,)), ((), ()))


def _split_bf16(x):
    hi = x.astype(BF16)
    return hi, (x - hi.astype(F32)).astype(BF16)


def _dot3_nn(x, w_hi, w_lo):
    xh, xl = _split_bf16(x)
    dn = (((1,), (0,)), ((), ()))
    return (jax.lax.dot_general(xh, w_hi, dn, preferred_element_type=F32)
            + jax.lax.dot_general(xh, w_lo, dn, preferred_element_type=F32)
            + jax.lax.dot_general(xl, w_hi, dn, preferred_element_type=F32))


def _dot3_nt(x, w_hi, w_lo):
    xh, xl = _split_bf16(x)
    return (jax.lax.dot_general(xh, w_hi, _DN_NT, preferred_element_type=F32)
            + jax.lax.dot_general(xh, w_lo, _DN_NT, preferred_element_type=F32)
            + jax.lax.dot_general(xl, w_hi, _DN_NT, preferred_element_type=F32))


def _gat_pre_body(x_ref, w1_ref, as1_ref, ad1_ref, sum8_ref,
                  h_ref, als_ref, ald_ref):
    h = jnp.dot(x_ref[...], w1_ref[...], preferred_element_type=F32)
    h_ref[...] = h
    als = jnp.dot(h * as1_ref[...], sum8_ref[...], preferred_element_type=F32)
    ald = jnp.dot(h * ad1_ref[...], sum8_ref[...], preferred_element_type=F32)
    als_ref[...] = jnp.concatenate([als, als], axis=1)
    ald_ref[...] = jnp.concatenate([ald, ald], axis=1)


def _gat_mid_body(agg_ref, b1_ref, w2_ref, as2_ref, ad2_ref, sum8_ref,
                  h_ref, als_ref, ald_ref):
    o1 = agg_ref[...] + b1_ref[...]
    x1 = jnp.where(o1 > 0.0, o1, jnp.exp(jnp.minimum(o1, 0.0)) - 1.0)
    h = jnp.dot(x1, w2_ref[...], preferred_element_type=F32)
    h_ref[...] = h
    als = jnp.dot(h * as2_ref[...], sum8_ref[...], preferred_element_type=F32)
    ald = jnp.dot(h * ad2_ref[...], sum8_ref[...], preferred_element_type=F32)
    als_ref[...] = jnp.concatenate([als, als], axis=1)
    ald_ref[...] = jnp.concatenate([ald, ald], axis=1)


def _gat_post_body(agg_ref, avg_ref, b2_ref, out_ref):
    out_ref[...] = jnp.dot(agg_ref[...], avg_ref[...], preferred_element_type=F32) + b2_ref[...]


def _sc_agg_body(h_hbm, als_hbm, ald_hbm, idx_hbm, out_hbm,
                 h_v, als_v, ald_v, idx_v, coef_v, den_v, out_v,
                 s_sm, t_sm, a_sm):
    wid = jax.lax.axis_index("s") * 2 + jax.lax.axis_index("c")

    @pl.when(wid < G)
    def _():
        base = wid * GP
        pltpu.sync_copy(h_hbm.at[pl.ds(base, GP)], h_v)
        pltpu.sync_copy(als_hbm.at[pl.ds(base, GP)], als_v)
        pltpu.sync_copy(ald_hbm.at[pl.ds(base, GP)], ald_v)
        pltpu.sync_copy(idx_hbm, idx_v)

        for k in range(3):
            sv = idx_v[0, pl.ds(k * 16, 16)]
            tv = idx_v[1, pl.ds(k * 16, 16)]
            for j in range(16):
                e = k * 16 + j
                if e < E:
                    s_sm[e] = sv[j]
                    t_sm[e] = tv[j]

        def logit_body(e, m):
            s = s_sm[e]
            t = t_sm[e]
            le = als_v[s] + ald_v[t]
            le = jnp.maximum(le, 0.2 * le)
            coef_v[e] = le
            return jnp.maximum(m, le)
        m16 = jax.lax.fori_loop(0, E, logit_body, jnp.full((16,), -1e30, F32))

        def zden_body(v, c):
            den_v[v] = jnp.zeros((16,), F32)
            return c
        jax.lax.fori_loop(0, V, zden_body, 0)

        def exp_body(e, c):
            t = t_sm[e]
            ex = jnp.exp(coef_v[e] - m16)
            coef_v[e] = ex
            den_v[t] = den_v[t] + ex
            return c
        jax.lax.fori_loop(0, E, exp_body, 0)

        def norm_body(e, c):
            t = t_sm[e]
            a = coef_v[e] / (den_v[t] + 1e-16)
            for h in range(HEADS):
                a_sm[e * HEADS + h] = a[h]
            return c
        jax.lax.fori_loop(0, E, norm_body, 0)

        def zout_body(v, c):
            for ch in range(HID // 16):
                out_v[v, pl.ds(ch * 16, 16)] = jnp.zeros((16,), F32)
            return c
        jax.lax.fori_loop(0, GP, zout_body, 0)

        def agg_body(e, c):
            s = s_sm[e]
            t = t_sm[e]
            for h in range(HEADS):
                a = a_sm[e * HEADS + h]
                for ch in range(HC // 16):
                    sl = pl.ds(h * HC + ch * 16, 16)
                    out_v[t, sl] = out_v[t, sl] + a * h_v[s, sl]
            return c
        jax.lax.fori_loop(0, E, agg_body, 0)

        pltpu.sync_copy(out_v, out_hbm.at[pl.ds(base, GP)])


@functools.cache
def _sc_agg():
    return pl.kernel(
        _sc_agg_body,
        out_type=jax.ShapeDtypeStruct((NPV, HID), F32),
        mesh=plsc.VectorSubcoreMesh(core_axis_name="c", subcore_axis_name="s"),
        scratch_types=[
            pltpu.VMEM((GP, HID), F32),
            pltpu.VMEM((GP, 16), F32),
            pltpu.VMEM((GP, 16), F32),
            pltpu.VMEM((2, 48), jnp.int32),
            pltpu.VMEM((48, 16), F32),
            pltpu.VMEM((V, 16), F32),
            pltpu.VMEM((GP, HID), F32),
            pltpu.SMEM((48,), jnp.int32),
            pltpu.SMEM((48,), jnp.int32),
            pltpu.SMEM((E * HEADS + 16,), F32),
        ],
    )


def _edge_body(px_ref, py_ref, d0_ref, d1_ref, wa_ref, wb_ref, bfe_ref, out_ref):
    px, py = px_ref[...], py_ref[...]
    for r, d_ref in ((0, d0_ref), (1, d1_ref)):
        vx = jnp.dot(px, d_ref[...], preferred_element_type=F32, precision=HI)
        vy = jnp.dot(py, d_ref[...], preferred_element_type=F32, precision=HI)
        ln = jnp.sqrt(vx * vx + vy * vy)
        ang = jnp.arctan2(vy, vx)
        o = (jnp.dot(ln, wa_ref[...], preferred_element_type=F32)
             + jnp.dot(ang, wb_ref[...], preferred_element_type=F32)
             + bfe_ref[...])
        out_ref[:, r * HC:(r + 1) * HC] = o


def _proj_body(xe_ref, scv_ref, shv_ref, xg_ref, scg_ref, shg_ref, sel_ref,
               wte_ref, wtg_ref, b_ref, out_ref):
    xg = xg_ref[...] * scg_ref[...] + shg_ref[...]
    g36 = jax.lax.dot_general(xg, wtg_ref[...], _DN_NT, preferred_element_type=F32)
    xe = xe_ref[...] * scv_ref[...] + shv_ref[...]
    out_ref[...] = (jax.lax.dot_general(xe, wte_ref[...], _DN_NT, preferred_element_type=F32)
                    + jnp.dot(sel_ref[...], g36, preferred_element_type=F32)
                    + b_ref[0])


def _lstm_body(g_ref, whfh_ref, whfl_ref, whbh_ref, whbl_ref, watt_ref,
               wcls_ref, bcls_ref, att_ref, cls_ref, lo_ref):
    nb = 2 * B

    def cell(g):
        i = jax.nn.sigmoid(g[:, 0:HL])
        f = jax.nn.sigmoid(g[:, HL:2 * HL])
        gg = jnp.tanh(g[:, 2 * HL:3 * HL])
        o = jax.nn.sigmoid(g[:, 3 * HL:4 * HL])
        return i, f, gg, o

    hf = jnp.zeros((nb, HL), F32)
    cf = jnp.zeros((nb, HL), F32)
    hb = jnp.zeros((nb, HL), F32)
    cb = jnp.zeros((nb, HL), F32)
    for t in range(T):
        gf = g_ref[t, :, 0:4 * HL] + _dot3_nt(hf, whfh_ref[...], whfl_ref[...])
        i, f, gg, o = cell(gf)
        cf = f * cf + i * gg
        hf = o * jnp.tanh(cf)
        lo_ref[t, :, 0:HL] = hf
        tb = T - 1 - t
        gb = g_ref[tb, :, 4 * HL:8 * HL] + _dot3_nt(hb, whbh_ref[...], whbl_ref[...])
        i, f, gg, o = cell(gb)
        cb = f * cb + i * gg
        hb = o * jnp.tanh(cb)
        lo_ref[tb, :, HL:2 * HL] = hb

    scores = jnp.concatenate(
        [jnp.dot(lo_ref[t], watt_ref[...], preferred_element_type=F32)
         for t in range(T)], axis=1)
    m = jnp.max(scores, axis=1, keepdims=True)
    e = jnp.exp(scores - m)
    aw = e / jnp.sum(e, axis=1, keepdims=True)
    att = jnp.zeros((nb, 2 * HL), F32)
    for t in range(T):
        att = att + aw[:, t:t + 1] * lo_ref[t]
    att_ref[...] = att
    cls_ref[...] = jnp.dot(att, wcls_ref[...], preferred_element_type=F32) + bcls_ref[...]


def kernel(pose1, pose2, connections, W1, att_src1, att_dst1, b1, W2, att_src2,
           att_dst2, b2, Wfe, bfe, bn_gamma, bn_beta, bn_mean, bn_var, Wih_f,
           Whh_f, bih_f, bhh_f, Wih_b, Whh_b, bih_b, bhh_b, Watt, batt, Wcls,
           bcls):
    s_idx = connections[0].astype(jnp.int32)
    t_idx = connections[1].astype(jnp.int32)

    sum8 = jax.nn.one_hot(jnp.arange(HID, dtype=jnp.int32) // HC, HEADS, dtype=F32)
    avg8 = jax.nn.one_hot(jnp.arange(HID, dtype=jnp.int32) % HC, HC, dtype=F32) / HEADS
    idx48 = jnp.zeros((2, 48), jnp.int32).at[0, :E].set(s_idx).at[1, :E].set(t_idx)

    x24 = jnp.concatenate([pose1[0], pose2[0]], axis=0).reshape(G, V, 3)
    x_pad = jnp.zeros((G, GP, 3), F32).at[:, :V].set(x24).reshape(NPV, 3)
    h1, als1, ald1 = pl.pallas_call(
        _gat_pre_body,
        out_shape=(jax.ShapeDtypeStruct((NPV, HID), F32),
                   jax.ShapeDtypeStruct((NPV, 16), F32),
                   jax.ShapeDtypeStruct((NPV, 16), F32)),
    )(x_pad, W1, att_src1.reshape(1, HID), att_dst1.reshape(1, HID), sum8)
    agg1 = _sc_agg()(h1, als1, ald1, idx48)
    h2, als2, ald2 = pl.pallas_call(
        _gat_mid_body,
        out_shape=(jax.ShapeDtypeStruct((NPV, HID), F32),
                   jax.ShapeDtypeStruct((NPV, 16), F32),
                   jax.ShapeDtypeStruct((NPV, 16), F32)),
    )(agg1, b1.reshape(1, HID), W2, att_src2.reshape(1, HID),
      att_dst2.reshape(1, HID), sum8)
    agg2 = _sc_agg()(h2, als2, ald2, idx48)
    gat_nodes = pl.pallas_call(
        _gat_post_body,
        out_shape=jax.ShapeDtypeStruct((NPV, HC), F32),
    )(agg2, avg8, b2.reshape(1, HC))

    pall = jnp.concatenate([pose1, pose2], axis=0).transpose(1, 0, 2, 3)
    pall = pall.reshape(2 * B * T, V, 3)
    px, py = pall[:, :, 0], pall[:, :, 1]
    dmat = (jax.nn.one_hot(t_idx, V, dtype=F32) - jax.nn.one_hot(s_idx, V, dtype=F32)).T
    edge_out = pl.pallas_call(
        _edge_body,
        out_shape=jax.ShapeDtypeStruct((2 * B * T, 2 * HC), F32),
    )(px, py, dmat[:, :E // 2], dmat[:, E // 2:], Wfe[0::2], Wfe[1::2],
      bfe.reshape(1, HC))

    sc = bn_gamma / jnp.sqrt(bn_var + 1e-5)
    sh = bn_beta - bn_mean * sc
    gat2 = gat_nodes.reshape(G, GP, HC)[:, :V].reshape(G, V * HC)
    dead = jnp.tile(b2, V)
    xg36 = jnp.concatenate(
        [jnp.broadcast_to(dead, (T, V * HC)), gat2], axis=0)
    scg = jnp.tile(sc, 3).reshape(3 * T, 1)
    shg = jnp.tile(sh, 3).reshape(3 * T, 1)
    tcol = jnp.arange(2 * B * T, dtype=jnp.int32) // (2 * B)
    bcol = jnp.arange(2 * B * T, dtype=jnp.int32) % (2 * B)
    sel_idx = jnp.where(bcol == 0, 12 + tcol,
                        jnp.where(bcol == B, 12 + T + tcol, tcol))
    sel768 = jax.nn.one_hot(sel_idx, 3 * T, dtype=F32)

    scv = jnp.repeat(sc, 2 * B).reshape(2 * B * T, 1)
    shv = jnp.repeat(sh, 2 * B).reshape(2 * B * T, 1)
    wt = jnp.concatenate([Wih_f, Wih_b], axis=0)
    wte = wt[:, V * HC:]
    wtg = wt[:, :V * HC]
    bias = jnp.concatenate([bih_f + bhh_f, bih_b + bhh_b]).reshape(8, 1, HL)
    nblk = 8
    proj = pl.pallas_call(
        _proj_body,
        grid=(nblk,),
        in_specs=[
            pl.BlockSpec((2 * B * T, 2 * HC), lambda i: (0, 0)),
            pl.BlockSpec((2 * B * T, 1), lambda i: (0, 0)),
            pl.BlockSpec((2 * B * T, 1), lambda i: (0, 0)),
            pl.BlockSpec((3 * T, V * HC), lambda i: (0, 0)),
            pl.BlockSpec((3 * T, 1), lambda i: (0, 0)),
            pl.BlockSpec((3 * T, 1), lambda i: (0, 0)),
            pl.BlockSpec((2 * B * T, 3 * T), lambda i: (0, 0)),
            pl.BlockSpec((HL, 2 * HC), lambda i: (i, 0)),
            pl.BlockSpec((HL, V * HC), lambda i: (i, 0)),
            pl.BlockSpec((1, 1, HL), lambda i: (i, 0, 0)),
        ],
        out_specs=pl.BlockSpec((2 * B * T, HL), lambda i: (0, i)),
        out_shape=jax.ShapeDtypeStruct((2 * B * T, 8 * HL), F32),
    )(edge_out, scv, shv, xg36, scg, shg, sel768, wte, wtg, bias)

    whfh, whfl = _split_bf16(Whh_f)
    whbh, whbl = _split_bf16(Whh_b)
    att, cls = pl.pallas_call(
        _lstm_body,
        out_shape=(jax.ShapeDtypeStruct((2 * B, 2 * HL), F32),
                   jax.ShapeDtypeStruct((2 * B, NCLS), F32)),
        scratch_shapes=[pltpu.VMEM((T, 2 * B, 2 * HL), F32)],
    )(proj.reshape(T, 2 * B, 8 * HL), whfh, whfl, whbh, whbl, Watt, Wcls,
      bcls.reshape(1, NCLS))
    return att, cls


# speedup vs baseline: 1.6168x; 1.1888x over previous
import functools

import jax
import jax.numpy as jnp
from jax.experimental import pallas as pl
from jax.experimental.pallas import tpu as pltpu
from jax.experimental.pallas import tpu_sc as plsc

B, T, V, E = 32, 12, 17, 38
HEADS, HC, HL, NCLS = 8, 128, 512, 500
G = 2 * T
NGV = G * V
NGE = G * E
HID = HEADS * HC
D = HC * (V + 2)
GP = 24
NPV = G * GP
F32 = jnp.float32
HI = jax.lax.Precision.HIGHEST
BF16 = jnp.bfloat16
_DN_NT = (((1,), (1,)), ((), ()))


def _split_bf16(x):
    hi = x.astype(BF16)
    return hi, (x - hi.astype(F32)).astype(BF16)


def _dot3_nn(x, w_hi, w_lo):
    xh, xl = _split_bf16(x)
    dn = (((1,), (0,)), ((), ()))
    return (jax.lax.dot_general(xh, w_hi, dn, preferred_element_type=F32)
            + jax.lax.dot_general(xh, w_lo, dn, preferred_element_type=F32)
            + jax.lax.dot_general(xl, w_hi, dn, preferred_element_type=F32))


def _dot3_nt(x, w_hi, w_lo):
    xh, xl = _split_bf16(x)
    return (jax.lax.dot_general(xh, w_hi, _DN_NT, preferred_element_type=F32)
            + jax.lax.dot_general(xh, w_lo, _DN_NT, preferred_element_type=F32)
            + jax.lax.dot_general(xl, w_hi, _DN_NT, preferred_element_type=F32))


def _gat_pre_body(x_ref, w1_ref, as1_ref, ad1_ref, sum8_ref,
                  h_ref, als_ref, ald_ref):
    h = jnp.dot(x_ref[...], w1_ref[...], preferred_element_type=F32)
    h_ref[...] = h
    als = jnp.dot(h * as1_ref[...], sum8_ref[...], preferred_element_type=F32)
    ald = jnp.dot(h * ad1_ref[...], sum8_ref[...], preferred_element_type=F32)
    als_ref[...] = jnp.concatenate([als, als], axis=1)
    ald_ref[...] = jnp.concatenate([ald, ald], axis=1)


def _gat_mid_body(agg_ref, b1_ref, w2_ref, as2_ref, ad2_ref, sum8_ref,
                  h_ref, als_ref, ald_ref):
    o1 = agg_ref[...] + b1_ref[...]
    x1 = jnp.where(o1 > 0.0, o1, jnp.exp(jnp.minimum(o1, 0.0)) - 1.0)
    h = jnp.dot(x1, w2_ref[...], preferred_element_type=F32)
    h_ref[...] = h
    als = jnp.dot(h * as2_ref[...], sum8_ref[...], preferred_element_type=F32)
    ald = jnp.dot(h * ad2_ref[...], sum8_ref[...], preferred_element_type=F32)
    als_ref[...] = jnp.concatenate([als, als], axis=1)
    ald_ref[...] = jnp.concatenate([ald, ald], axis=1)


def _gat_post_body(agg_ref, avg_ref, b2_ref, out_ref):
    out_ref[...] = jnp.dot(agg_ref[...], avg_ref[...], preferred_element_type=F32) + b2_ref[...]


def _sc_agg_body(h_hbm, als_hbm, ald_hbm, idx_hbm, out_hbm,
                 h_v, als_v, ald_v, idx_v, coef_v, den_v, out_v,
                 s_sm, t_sm, a_sm):
    wid = jax.lax.axis_index("s") * 2 + jax.lax.axis_index("c")

    @pl.when(wid < G)
    def _():
        base = wid * GP
        pltpu.sync_copy(h_hbm.at[pl.ds(base, GP)], h_v)
        pltpu.sync_copy(als_hbm.at[pl.ds(base, GP)], als_v)
        pltpu.sync_copy(ald_hbm.at[pl.ds(base, GP)], ald_v)
        pltpu.sync_copy(idx_hbm, idx_v)

        for k in range(3):
            sv = idx_v[0, pl.ds(k * 16, 16)]
            tv = idx_v[1, pl.ds(k * 16, 16)]
            for j in range(16):
                e = k * 16 + j
                if e < E:
                    s_sm[e] = sv[j]
                    t_sm[e] = tv[j]

        def logit_body(e, m):
            s = s_sm[e]
            t = t_sm[e]
            le = als_v[s] + ald_v[t]
            le = jnp.maximum(le, 0.2 * le)
            coef_v[e] = le
            return jnp.maximum(m, le)
        m16 = jax.lax.fori_loop(0, E, logit_body, jnp.full((16,), -1e30, F32))

        def zden_body(v, c):
            den_v[v] = jnp.zeros((16,), F32)
            return c
        jax.lax.fori_loop(0, V, zden_body, 0)

        def exp_body(e, c):
            t = t_sm[e]
            ex = jnp.exp(coef_v[e] - m16)
            coef_v[e] = ex
            den_v[t] = den_v[t] + ex
            return c
        jax.lax.fori_loop(0, E, exp_body, 0)

        def norm_body(e, c):
            t = t_sm[e]
            a = coef_v[e] / (den_v[t] + 1e-16)
            for h in range(HEADS):
                a_sm[e * HEADS + h] = a[h]
            return c
        jax.lax.fori_loop(0, E, norm_body, 0)

        def zout_body(v, c):
            for ch in range(HID // 16):
                out_v[v, pl.ds(ch * 16, 16)] = jnp.zeros((16,), F32)
            return c
        jax.lax.fori_loop(0, GP, zout_body, 0)

        def agg_body(e, c):
            s = s_sm[e]
            t = t_sm[e]
            for h in range(HEADS):
                a = a_sm[e * HEADS + h]
                for ch in range(HC // 16):
                    sl = pl.ds(h * HC + ch * 16, 16)
                    out_v[t, sl] = out_v[t, sl] + a * h_v[s, sl]
            return c
        jax.lax.fori_loop(0, E, agg_body, 0)

        pltpu.sync_copy(out_v, out_hbm.at[pl.ds(base, GP)])


@functools.cache
def _sc_agg():
    return pl.kernel(
        _sc_agg_body,
        out_type=jax.ShapeDtypeStruct((NPV, HID), F32),
        mesh=plsc.VectorSubcoreMesh(core_axis_name="c", subcore_axis_name="s"),
        scratch_types=[
            pltpu.VMEM((GP, HID), F32),
            pltpu.VMEM((GP, 16), F32),
            pltpu.VMEM((GP, 16), F32),
            pltpu.VMEM((2, 48), jnp.int32),
            pltpu.VMEM((48, 16), F32),
            pltpu.VMEM((V, 16), F32),
            pltpu.VMEM((GP, HID), F32),
            pltpu.SMEM((48,), jnp.int32),
            pltpu.SMEM((48,), jnp.int32),
            pltpu.SMEM((E * HEADS + 16,), F32),
        ],
    )


def _edge_body(px_ref, py_ref, d0_ref, d1_ref, wa_ref, wb_ref, bfe_ref, out_ref):
    px, py = px_ref[...], py_ref[...]
    for r, d_ref in ((0, d0_ref), (1, d1_ref)):
        vx = jnp.dot(px, d_ref[...], preferred_element_type=F32, precision=HI)
        vy = jnp.dot(py, d_ref[...], preferred_element_type=F32, precision=HI)
        ln = jnp.sqrt(vx * vx + vy * vy)
        ang = jnp.arctan2(vy, vx)
        o = (jnp.dot(ln, wa_ref[...], preferred_element_type=F32)
             + jnp.dot(ang, wb_ref[...], preferred_element_type=F32)
             + bfe_ref[...])
        out_ref[:, r * HC:(r + 1) * HC] = o


def _proj_body(xe_ref, scv_ref, shv_ref, xg_ref, scg_ref, shg_ref, sel_ref,
               wte_ref, wtg_ref, b_ref, out_ref):
    xg = xg_ref[...] * scg_ref[...] + shg_ref[...]
    g36 = jax.lax.dot_general(xg, wtg_ref[...], _DN_NT, preferred_element_type=F32)
    xe = xe_ref[...] * scv_ref[...] + shv_ref[...]
    out_ref[...] = (jax.lax.dot_general(xe, wte_ref[...], _DN_NT, preferred_element_type=F32)
                    + jnp.dot(sel_ref[...], g36, preferred_element_type=F32)
                    + b_ref[0])


def _lstm_body(g_ref, whfh_ref, whfl_ref, whbh_ref, whbl_ref, watt_ref,
               wcls_ref, bcls_ref, att_ref, cls_ref, lo_ref):
    nb = 2 * B

    def cell(g):
        i = jax.nn.sigmoid(g[:, 0:HL])
        f = jax.nn.sigmoid(g[:, HL:2 * HL])
        gg = jnp.tanh(g[:, 2 * HL:3 * HL])
        o = jax.nn.sigmoid(g[:, 3 * HL:4 * HL])
        return i, f, gg, o

    hf = jnp.zeros((nb, HL), F32)
    cf = jnp.zeros((nb, HL), F32)
    hb = jnp.zeros((nb, HL), F32)
    cb = jnp.zeros((nb, HL), F32)
    for t in range(T):
        gf = g_ref[t, :, 0:4 * HL] + jax.lax.dot_general(hf.astype(BF16), whfh_ref[...], _DN_NT, preferred_element_type=F32)
        i, f, gg, o = cell(gf)
        cf = f * cf + i * gg
        hf = o * jnp.tanh(cf)
        lo_ref[t, :, 0:HL] = hf
        tb = T - 1 - t
        gb = g_ref[tb, :, 4 * HL:8 * HL] + jax.lax.dot_general(hb.astype(BF16), whbh_ref[...], _DN_NT, preferred_element_type=F32)
        i, f, gg, o = cell(gb)
        cb = f * cb + i * gg
        hb = o * jnp.tanh(cb)
        lo_ref[tb, :, HL:2 * HL] = hb

    scores = jnp.concatenate(
        [jnp.dot(lo_ref[t], watt_ref[...], preferred_element_type=F32)
         for t in range(T)], axis=1)
    m = jnp.max(scores, axis=1, keepdims=True)
    e = jnp.exp(scores - m)
    aw = e / jnp.sum(e, axis=1, keepdims=True)
    att = jnp.zeros((nb, 2 * HL), F32)
    for t in range(T):
        att = att + aw[:, t:t + 1] * lo_ref[t]
    att_ref[...] = att
    cls_ref[...] = jnp.dot(att, wcls_ref[...], preferred_element_type=F32) + bcls_ref[...]


def kernel(pose1, pose2, connections, W1, att_src1, att_dst1, b1, W2, att_src2,
           att_dst2, b2, Wfe, bfe, bn_gamma, bn_beta, bn_mean, bn_var, Wih_f,
           Whh_f, bih_f, bhh_f, Wih_b, Whh_b, bih_b, bhh_b, Watt, batt, Wcls,
           bcls):
    s_idx = connections[0].astype(jnp.int32)
    t_idx = connections[1].astype(jnp.int32)

    sum8 = jax.nn.one_hot(jnp.arange(HID, dtype=jnp.int32) // HC, HEADS, dtype=F32)
    avg8 = jax.nn.one_hot(jnp.arange(HID, dtype=jnp.int32) % HC, HC, dtype=F32) / HEADS
    idx48 = jnp.zeros((2, 48), jnp.int32).at[0, :E].set(s_idx).at[1, :E].set(t_idx)

    x24 = jnp.concatenate([pose1[0], pose2[0]], axis=0).reshape(G, V, 3)
    x_pad = jnp.zeros((G, GP, 3), F32).at[:, :V].set(x24).reshape(NPV, 3)
    h1, als1, ald1 = pl.pallas_call(
        _gat_pre_body,
        out_shape=(jax.ShapeDtypeStruct((NPV, HID), F32),
                   jax.ShapeDtypeStruct((NPV, 16), F32),
                   jax.ShapeDtypeStruct((NPV, 16), F32)),
    )(x_pad, W1, att_src1.reshape(1, HID), att_dst1.reshape(1, HID), sum8)
    agg1 = _sc_agg()(h1, als1, ald1, idx48)
    h2, als2, ald2 = pl.pallas_call(
        _gat_mid_body,
        out_shape=(jax.ShapeDtypeStruct((NPV, HID), F32),
                   jax.ShapeDtypeStruct((NPV, 16), F32),
                   jax.ShapeDtypeStruct((NPV, 16), F32)),
    )(agg1, b1.reshape(1, HID), W2, att_src2.reshape(1, HID),
      att_dst2.reshape(1, HID), sum8)
    agg2 = _sc_agg()(h2, als2, ald2, idx48)
    gat_nodes = pl.pallas_call(
        _gat_post_body,
        out_shape=jax.ShapeDtypeStruct((NPV, HC), F32),
    )(agg2, avg8, b2.reshape(1, HC))

    pall = jnp.concatenate([pose1, pose2], axis=0).transpose(1, 0, 2, 3)
    pall = pall.reshape(2 * B * T, V, 3)
    px, py = pall[:, :, 0], pall[:, :, 1]
    dmat = (jax.nn.one_hot(t_idx, V, dtype=F32) - jax.nn.one_hot(s_idx, V, dtype=F32)).T
    edge_out = pl.pallas_call(
        _edge_body,
        out_shape=jax.ShapeDtypeStruct((2 * B * T, 2 * HC), F32),
    )(px, py, dmat[:, :E // 2], dmat[:, E // 2:], Wfe[0::2], Wfe[1::2],
      bfe.reshape(1, HC))

    sc = bn_gamma / jnp.sqrt(bn_var + 1e-5)
    sh = bn_beta - bn_mean * sc
    gat2 = gat_nodes.reshape(G, GP, HC)[:, :V].reshape(G, V * HC)
    dead = jnp.tile(b2, V)
    xg36 = jnp.concatenate(
        [jnp.broadcast_to(dead, (T, V * HC)), gat2], axis=0)
    scg = jnp.tile(sc, 3).reshape(3 * T, 1)
    shg = jnp.tile(sh, 3).reshape(3 * T, 1)
    tcol = jnp.arange(2 * B * T, dtype=jnp.int32) // (2 * B)
    bcol = jnp.arange(2 * B * T, dtype=jnp.int32) % (2 * B)
    sel_idx = jnp.where(bcol == 0, 12 + tcol,
                        jnp.where(bcol == B, 12 + T + tcol, tcol))
    sel768 = jax.nn.one_hot(sel_idx, 3 * T, dtype=F32)

    scv = jnp.repeat(sc, 2 * B).reshape(2 * B * T, 1)
    shv = jnp.repeat(sh, 2 * B).reshape(2 * B * T, 1)
    wt = jnp.concatenate([Wih_f, Wih_b], axis=0)
    wte = wt[:, V * HC:]
    wtg = wt[:, :V * HC]
    bias = jnp.concatenate([bih_f + bhh_f, bih_b + bhh_b]).reshape(8, 1, HL)
    nblk = 8
    proj = pl.pallas_call(
        _proj_body,
        grid=(nblk,),
        in_specs=[
            pl.BlockSpec((2 * B * T, 2 * HC), lambda i: (0, 0)),
            pl.BlockSpec((2 * B * T, 1), lambda i: (0, 0)),
            pl.BlockSpec((2 * B * T, 1), lambda i: (0, 0)),
            pl.BlockSpec((3 * T, V * HC), lambda i: (0, 0)),
            pl.BlockSpec((3 * T, 1), lambda i: (0, 0)),
            pl.BlockSpec((3 * T, 1), lambda i: (0, 0)),
            pl.BlockSpec((2 * B * T, 3 * T), lambda i: (0, 0)),
            pl.BlockSpec((HL, 2 * HC), lambda i: (i, 0)),
            pl.BlockSpec((HL, V * HC), lambda i: (i, 0)),
            pl.BlockSpec((1, 1, HL), lambda i: (i, 0, 0)),
        ],
        out_specs=pl.BlockSpec((2 * B * T, HL), lambda i: (0, i)),
        out_shape=jax.ShapeDtypeStruct((2 * B * T, 8 * HL), F32),
    )(edge_out, scv, shv, xg36, scg, shg, sel768, wte, wtg, bias)

    whfh, whfl = _split_bf16(Whh_f)
    whbh, whbl = _split_bf16(Whh_b)
    att, cls = pl.pallas_call(
        _lstm_body,
        out_shape=(jax.ShapeDtypeStruct((2 * B, 2 * HL), F32),
                   jax.ShapeDtypeStruct((2 * B, NCLS), F32)),
        scratch_shapes=[pltpu.VMEM((T, 2 * B, 2 * HL), F32)],
    )(proj.reshape(T, 2 * B, 8 * HL), whfh, whfl, whbh, whbl, Watt, Wcls,
      bcls.reshape(1, NCLS))
    return att, cls


# speedup vs baseline: 1.6230x; 1.0039x over previous
import functools

import jax
import jax.numpy as jnp
from jax.experimental import pallas as pl
from jax.experimental.pallas import tpu as pltpu
from jax.experimental.pallas import tpu_sc as plsc

B, T, V, E = 32, 12, 17, 38
HEADS, HC, HL, NCLS = 8, 128, 512, 500
G = 2 * T
NGV = G * V
NGE = G * E
HID = HEADS * HC
D = HC * (V + 2)
GP = 24
NPV = G * GP
F32 = jnp.float32
HI = jax.lax.Precision.HIGHEST
BF16 = jnp.bfloat16
_DN_NT = (((1,), (1,)), ((), ()))


def _gat_pre_body(x_ref, w1_ref, as1_ref, ad1_ref, sum8_ref,
                  h_ref, als_ref, ald_ref):
    h = jnp.dot(x_ref[...], w1_ref[...], preferred_element_type=F32)
    h_ref[...] = h
    als = jnp.dot(h * as1_ref[...], sum8_ref[...], preferred_element_type=F32)
    ald = jnp.dot(h * ad1_ref[...], sum8_ref[...], preferred_element_type=F32)
    als_ref[...] = jnp.concatenate([als, als], axis=1)
    ald_ref[...] = jnp.concatenate([ald, ald], axis=1)


def _gat_mid_body(agg_ref, b1_ref, w2_ref, as2_ref, ad2_ref, sum8_ref,
                  h_ref, als_ref, ald_ref):
    o1 = agg_ref[...] + b1_ref[...]
    x1 = jnp.where(o1 > 0.0, o1, jnp.exp(jnp.minimum(o1, 0.0)) - 1.0)
    h = jnp.dot(x1, w2_ref[...], preferred_element_type=F32)
    h_ref[...] = h
    als = jnp.dot(h * as2_ref[...], sum8_ref[...], preferred_element_type=F32)
    ald = jnp.dot(h * ad2_ref[...], sum8_ref[...], preferred_element_type=F32)
    als_ref[...] = jnp.concatenate([als, als], axis=1)
    ald_ref[...] = jnp.concatenate([ald, ald], axis=1)


def _gat_post_body(agg_ref, avg_ref, b2_ref, out_ref):
    out_ref[...] = jnp.dot(agg_ref[...], avg_ref[...], preferred_element_type=F32) + b2_ref[...]


def _sc_agg_body(h_hbm, als_hbm, ald_hbm, idx_hbm, out_hbm,
                 h_v, als_v, ald_v, idx_v, coef_v, den_v, out_v,
                 s_sm, t_sm, a_sm):
    wid = jax.lax.axis_index("s") * 2 + jax.lax.axis_index("c")

    @pl.when(wid < G)
    def _():
        base = wid * GP
        pltpu.sync_copy(h_hbm.at[pl.ds(base, GP)], h_v)
        pltpu.sync_copy(als_hbm.at[pl.ds(base, GP)], als_v)
        pltpu.sync_copy(ald_hbm.at[pl.ds(base, GP)], ald_v)
        pltpu.sync_copy(idx_hbm, idx_v)

        for k in range(3):
            sv = idx_v[0, pl.ds(k * 16, 16)]
            tv = idx_v[1, pl.ds(k * 16, 16)]
            for j in range(16):
                e = k * 16 + j
                if e < E:
                    s_sm[e] = sv[j]
                    t_sm[e] = tv[j]

        def logit_body(e, m):
            s = s_sm[e]
            t = t_sm[e]
            le = als_v[s] + ald_v[t]
            le = jnp.maximum(le, 0.2 * le)
            coef_v[e] = le
            return jnp.maximum(m, le)
        m16 = jax.lax.fori_loop(0, E, logit_body, jnp.full((16,), -1e30, F32))

        def zden_body(v, c):
            den_v[v] = jnp.zeros((16,), F32)
            return c
        jax.lax.fori_loop(0, V, zden_body, 0)

        def exp_body(e, c):
            t = t_sm[e]
            ex = jnp.exp(coef_v[e] - m16)
            coef_v[e] = ex
            den_v[t] = den_v[t] + ex
            return c
        jax.lax.fori_loop(0, E, exp_body, 0)

        def norm_body(e, c):
            t = t_sm[e]
            a = coef_v[e] / (den_v[t] + 1e-16)
            for h in range(HEADS):
                a_sm[e * HEADS + h] = a[h]
            return c
        jax.lax.fori_loop(0, E, norm_body, 0)

        def zout_body(v, c):
            for ch in range(HID // 16):
                out_v[v, pl.ds(ch * 16, 16)] = jnp.zeros((16,), F32)
            return c
        jax.lax.fori_loop(0, GP, zout_body, 0)

        def agg_body(e, c):
            s = s_sm[e]
            t = t_sm[e]
            for h in range(HEADS):
                a = a_sm[e * HEADS + h]
                for ch in range(HC // 16):
                    sl = pl.ds(h * HC + ch * 16, 16)
                    out_v[t, sl] = out_v[t, sl] + a * h_v[s, sl]
            return c
        jax.lax.fori_loop(0, E, agg_body, 0)

        pltpu.sync_copy(out_v, out_hbm.at[pl.ds(base, GP)])


@functools.cache
def _sc_agg():
    return pl.kernel(
        _sc_agg_body,
        out_type=jax.ShapeDtypeStruct((NPV, HID), F32),
        mesh=plsc.VectorSubcoreMesh(core_axis_name="c", subcore_axis_name="s"),
        scratch_types=[
            pltpu.VMEM((GP, HID), F32),
            pltpu.VMEM((GP, 16), F32),
            pltpu.VMEM((GP, 16), F32),
            pltpu.VMEM((2, 48), jnp.int32),
            pltpu.VMEM((48, 16), F32),
            pltpu.VMEM((V, 16), F32),
            pltpu.VMEM((GP, HID), F32),
            pltpu.SMEM((48,), jnp.int32),
            pltpu.SMEM((48,), jnp.int32),
            pltpu.SMEM((E * HEADS + 16,), F32),
        ],
    )


def _edge_body(px_ref, py_ref, d0_ref, d1_ref, wa_ref, wb_ref, bfe_ref, out_ref):
    px, py = px_ref[...], py_ref[...]
    for r, d_ref in ((0, d0_ref), (1, d1_ref)):
        vx = jnp.dot(px, d_ref[...], preferred_element_type=F32, precision=HI)
        vy = jnp.dot(py, d_ref[...], preferred_element_type=F32, precision=HI)
        ln = jnp.sqrt(vx * vx + vy * vy)
        ang = jnp.arctan2(vy, vx)
        o = (jnp.dot(ln, wa_ref[...], preferred_element_type=F32)
             + jnp.dot(ang, wb_ref[...], preferred_element_type=F32)
             + bfe_ref[...])
        out_ref[:, r * HC:(r + 1) * HC] = o


def _proj_body(xe_ref, scv_ref, shv_ref, xg_ref, scg_ref, shg_ref, sel_ref,
               wte_ref, wtg_ref, b_ref, out_ref):
    xg = xg_ref[...] * scg_ref[...] + shg_ref[...]
    g36 = jax.lax.dot_general(xg, wtg_ref[...], _DN_NT, preferred_element_type=F32)
    xe = xe_ref[...] * scv_ref[...] + shv_ref[...]
    out_ref[...] = (jax.lax.dot_general(xe, wte_ref[...], _DN_NT, preferred_element_type=F32)
                    + jnp.dot(sel_ref[...], g36, preferred_element_type=F32)
                    + b_ref[0])


def _lstm_body(g_ref, whf_ref, whb_ref, watt_ref,
               wcls_ref, bcls_ref, att_ref, cls_ref, lo_ref):
    nb = 2 * B

    def cell(g):
        i = jax.nn.sigmoid(g[:, 0:HL])
        f = jax.nn.sigmoid(g[:, HL:2 * HL])
        gg = jnp.tanh(g[:, 2 * HL:3 * HL])
        o = jax.nn.sigmoid(g[:, 3 * HL:4 * HL])
        return i, f, gg, o

    hf = jnp.zeros((nb, HL), F32)
    cf = jnp.zeros((nb, HL), F32)
    hb = jnp.zeros((nb, HL), F32)
    cb = jnp.zeros((nb, HL), F32)
    for t in range(T):
        gf = g_ref[t, :, 0:4 * HL] + jax.lax.dot_general(hf.astype(BF16), whf_ref[...], _DN_NT, preferred_element_type=F32)
        i, f, gg, o = cell(gf)
        cf = f * cf + i * gg
        hf = o * jnp.tanh(cf)
        lo_ref[t, :, 0:HL] = hf
        tb = T - 1 - t
        gb = g_ref[tb, :, 4 * HL:8 * HL] + jax.lax.dot_general(hb.astype(BF16), whb_ref[...], _DN_NT, preferred_element_type=F32)
        i, f, gg, o = cell(gb)
        cb = f * cb + i * gg
        hb = o * jnp.tanh(cb)
        lo_ref[tb, :, HL:2 * HL] = hb

    scores = jnp.concatenate(
        [jnp.dot(lo_ref[t], watt_ref[...], preferred_element_type=F32)
         for t in range(T)], axis=1)
    m = jnp.max(scores, axis=1, keepdims=True)
    e = jnp.exp(scores - m)
    aw = e / jnp.sum(e, axis=1, keepdims=True)
    att = jnp.zeros((nb, 2 * HL), F32)
    for t in range(T):
        att = att + aw[:, t:t + 1] * lo_ref[t]
    att_ref[...] = att
    cls_ref[...] = jnp.dot(att, wcls_ref[...], preferred_element_type=F32) + bcls_ref[...]


def kernel(pose1, pose2, connections, W1, att_src1, att_dst1, b1, W2, att_src2,
           att_dst2, b2, Wfe, bfe, bn_gamma, bn_beta, bn_mean, bn_var, Wih_f,
           Whh_f, bih_f, bhh_f, Wih_b, Whh_b, bih_b, bhh_b, Watt, batt, Wcls,
           bcls):
    s_idx = connections[0].astype(jnp.int32)
    t_idx = connections[1].astype(jnp.int32)

    sum8 = jax.nn.one_hot(jnp.arange(HID, dtype=jnp.int32) // HC, HEADS, dtype=F32)
    avg8 = jax.nn.one_hot(jnp.arange(HID, dtype=jnp.int32) % HC, HC, dtype=F32) / HEADS
    idx48 = jnp.zeros((2, 48), jnp.int32).at[0, :E].set(s_idx).at[1, :E].set(t_idx)

    x24 = jnp.concatenate([pose1[0], pose2[0]], axis=0).reshape(G, V, 3)
    x_pad = jnp.zeros((G, GP, 3), F32).at[:, :V].set(x24).reshape(NPV, 3)
    h1, als1, ald1 = pl.pallas_call(
        _gat_pre_body,
        out_shape=(jax.ShapeDtypeStruct((NPV, HID), F32),
                   jax.ShapeDtypeStruct((NPV, 16), F32),
                   jax.ShapeDtypeStruct((NPV, 16), F32)),
    )(x_pad, W1, att_src1.reshape(1, HID), att_dst1.reshape(1, HID), sum8)
    agg1 = _sc_agg()(h1, als1, ald1, idx48)
    h2, als2, ald2 = pl.pallas_call(
        _gat_mid_body,
        out_shape=(jax.ShapeDtypeStruct((NPV, HID), F32),
                   jax.ShapeDtypeStruct((NPV, 16), F32),
                   jax.ShapeDtypeStruct((NPV, 16), F32)),
    )(agg1, b1.reshape(1, HID), W2, att_src2.reshape(1, HID),
      att_dst2.reshape(1, HID), sum8)
    agg2 = _sc_agg()(h2, als2, ald2, idx48)
    gat_nodes = pl.pallas_call(
        _gat_post_body,
        out_shape=jax.ShapeDtypeStruct((NPV, HC), F32),
    )(agg2, avg8, b2.reshape(1, HC))

    pall = jnp.concatenate([pose1, pose2], axis=0).transpose(1, 0, 2, 3)
    pall = pall.reshape(2 * B * T, V, 3)
    px, py = pall[:, :, 0], pall[:, :, 1]
    dmat = (jax.nn.one_hot(t_idx, V, dtype=F32) - jax.nn.one_hot(s_idx, V, dtype=F32)).T
    edge_out = pl.pallas_call(
        _edge_body,
        out_shape=jax.ShapeDtypeStruct((2 * B * T, 2 * HC), F32),
    )(px, py, dmat[:, :E // 2], dmat[:, E // 2:], Wfe[0::2], Wfe[1::2],
      bfe.reshape(1, HC))

    sc = bn_gamma / jnp.sqrt(bn_var + 1e-5)
    sh = bn_beta - bn_mean * sc
    gat2 = gat_nodes.reshape(G, GP, HC)[:, :V].reshape(G, V * HC)
    dead = jnp.tile(b2, V)
    xg36 = jnp.concatenate(
        [jnp.broadcast_to(dead, (T, V * HC)), gat2], axis=0)
    scg = jnp.tile(sc, 3).reshape(3 * T, 1)
    shg = jnp.tile(sh, 3).reshape(3 * T, 1)
    tcol = jnp.arange(2 * B * T, dtype=jnp.int32) // (2 * B)
    bcol = jnp.arange(2 * B * T, dtype=jnp.int32) % (2 * B)
    sel_idx = jnp.where(bcol == 0, 12 + tcol,
                        jnp.where(bcol == B, 12 + T + tcol, tcol))
    sel768 = jax.nn.one_hot(sel_idx, 3 * T, dtype=F32)

    scv = jnp.repeat(sc, 2 * B).reshape(2 * B * T, 1)
    shv = jnp.repeat(sh, 2 * B).reshape(2 * B * T, 1)
    wt = jnp.concatenate([Wih_f, Wih_b], axis=0)
    wte = wt[:, V * HC:]
    wtg = wt[:, :V * HC]
    bias = jnp.concatenate([bih_f + bhh_f, bih_b + bhh_b]).reshape(8, 1, HL)
    nblk = 8
    proj = pl.pallas_call(
        _proj_body,
        grid=(nblk,),
        in_specs=[
            pl.BlockSpec((2 * B * T, 2 * HC), lambda i: (0, 0)),
            pl.BlockSpec((2 * B * T, 1), lambda i: (0, 0)),
            pl.BlockSpec((2 * B * T, 1), lambda i: (0, 0)),
            pl.BlockSpec((3 * T, V * HC), lambda i: (0, 0)),
            pl.BlockSpec((3 * T, 1), lambda i: (0, 0)),
            pl.BlockSpec((3 * T, 1), lambda i: (0, 0)),
            pl.BlockSpec((2 * B * T, 3 * T), lambda i: (0, 0)),
            pl.BlockSpec((HL, 2 * HC), lambda i: (i, 0)),
            pl.BlockSpec((HL, V * HC), lambda i: (i, 0)),
            pl.BlockSpec((1, 1, HL), lambda i: (i, 0, 0)),
        ],
        out_specs=pl.BlockSpec((2 * B * T, HL), lambda i: (0, i)),
        out_shape=jax.ShapeDtypeStruct((2 * B * T, 8 * HL), F32),
    )(edge_out, scv, shv, xg36, scg, shg, sel768, wte, wtg, bias)


    att, cls = pl.pallas_call(
        _lstm_body,
        out_shape=(jax.ShapeDtypeStruct((2 * B, 2 * HL), F32),
                   jax.ShapeDtypeStruct((2 * B, NCLS), F32)),
        scratch_shapes=[pltpu.VMEM((T, 2 * B, 2 * HL), F32)],
    )(proj.reshape(T, 2 * B, 8 * HL), Whh_f.astype(BF16), Whh_b.astype(BF16), Watt, Wcls,
      bcls.reshape(1, NCLS))
    return att, cls
